# baseline (speedup 1.0000x reference)
import functools

import jax
import jax.numpy as jnp
from jax import lax
from jax.experimental import pallas as pl
from jax.experimental.pallas import tpu as pltpu
from jax.experimental.pallas import tpu_sc as plsc

N_ROWS = 16384
N_COLS = 4096
NC = 2
NS = 16
NW = NC * NS
R = N_ROWS // NW
L = 16
G = R // L
NCORNER = 16
CHUNK = 128
NCHUNK = R * NCORNER // CHUNK

COEF = (512, 64, 8, 1)
OFFS = tuple(
    (c >> 3 & 1) * 512 + (c >> 2 & 1) * 64 + (c >> 1 & 1) * 8 + (c & 1)
    for c in range(NCORNER)
)


def _interp_body(coords_hbm, mesh_hbm, out_hbm,
                 coords_v, cf_v, idx_v, vals_v, out_v, sem):
    cid = lax.axis_index("c")
    sid = lax.axis_index("s")
    wid = sid * NC + cid
    base_row = wid * R

    pltpu.sync_copy(coords_hbm.at[:, pl.ds(base_row, R)], coords_v)

    lane = lax.iota(jnp.int32, L)

    for g in range(G):
        o = g * L
        fb = (base_row + o + lane) * N_COLS
        for d in range(4):
            c = coords_v[d, pl.ds(o, L)] * 7.0
            ci = c.astype(jnp.int32)
            ci = jnp.maximum(ci, 0)
            ci = jnp.minimum(ci, 6)
            cf_v[d, pl.ds(o, L)] = c - ci.astype(jnp.float32)
            fb = fb + ci * COEF[d]
        for corner in range(NCORNER):
            p = corner * R + o
            idx_v[p // CHUNK, pl.ds(p % CHUNK, L)] = fb + OFFS[corner]

    copies = []
    for j in range(NCHUNK):
        copies.append(
            pltpu.async_copy(
                mesh_hbm.at[idx_v.at[j]],
                vals_v.at[pl.ds(j * CHUNK, CHUNK)],
                sem,
            )
        )
    for cp in copies:
        cp.wait()

    for g in range(G):
        o = g * L
        cf0 = cf_v[0, pl.ds(o, L)]
        cf1 = cf_v[1, pl.ds(o, L)]
        cf2 = cf_v[2, pl.ds(o, L)]
        cf3 = cf_v[3, pl.ds(o, L)]
        w0 = (1.0 - cf0, cf0)
        w1 = (1.0 - cf1, cf1)
        w2 = (1.0 - cf2, cf2)
        w3 = (1.0 - cf3, cf3)
        w01 = [[w0[a] * w1[b] for b in range(2)] for a in range(2)]
        w23 = [[w2[a] * w3[b] for b in range(2)] for a in range(2)]
        acc = None
        for corner in range(NCORNER):
            d0, d1, d2, d3 = corner >> 3 & 1, corner >> 2 & 1, corner >> 1 & 1, corner & 1
            v = vals_v[pl.ds(corner * R + o, L)]
            term = v * (w01[d0][d1] * w23[d2][d3])
            acc = term if acc is None else acc + term
        out_v[pl.ds(o, L)] = acc

    pltpu.sync_copy(out_v, out_hbm.at[pl.ds(base_row, R)])


_interp_kernel = functools.partial(
    pl.kernel,
    out_type=jax.ShapeDtypeStruct((N_ROWS,), jnp.float32),
    mesh=plsc.VectorSubcoreMesh(core_axis_name="c", subcore_axis_name="s"),
    scratch_types=[
        pltpu.VMEM((4, R), jnp.float32),
        pltpu.VMEM((4, R), jnp.float32),
        pltpu.VMEM((NCHUNK, CHUNK), jnp.int32),
        pltpu.VMEM((R * NCORNER,), jnp.float32),
        pltpu.VMEM((R,), jnp.float32),
        pltpu.SemaphoreType.DMA,
    ],
)(_interp_body)


def kernel(coordinates, mesh_pred):
    coords_t = coordinates.T.reshape(4, N_ROWS)
    mesh_flat = mesh_pred.reshape(N_ROWS * N_COLS)
    return _interp_kernel(coords_t, mesh_flat)
